# Initial kernel scaffold; baseline (speedup 1.0000x reference)
import jax
import jax.numpy as jnp
from jax.experimental import pallas as pl


def kernel(x, edge_index, batch, batch_size, W_in, b_in, eW1, eb1, eW2, eb2, nW1, nb1, mW0, mb0, mW1, mb1, mW2, mb2, mW3, mb3):
    raise NotImplementedError("write your pallas kernel here")


# speedup vs baseline: 3.3949x; 3.3949x over previous
import functools

import jax
import jax.numpy as jnp
from jax import lax
from jax.experimental import pallas as pl
from jax.experimental.pallas import tpu as pltpu
from jax.experimental.pallas import tpu_sc as plsc

N = 50000
E = 800000
H = 64
MH = 128
B = 1024
N_ITERS = 3

NC = 2
NS = 16
LANES = 16

F32 = jnp.float32
I32 = jnp.int32


def _ln(v, eps=1e-5):
    mu = jnp.mean(v, axis=-1, keepdims=True)
    var = jnp.mean((v - mu) * (v - mu), axis=-1, keepdims=True)
    return (v - mu) * lax.rsqrt(var + eps)


def _dot(a, b):
    return lax.dot_general(a, b, (((1,), (0,)), ((), ())),
                           preferred_element_type=F32)


_RN = 2000


def _h0_body(x_ref, w_ref, b_ref, o_ref):
    x = x_ref[...]
    w = w_ref[...]
    h = (x[:, 0:1] * w[0:1, :] + x[:, 1:2] * w[1:2, :]
         + x[:, 2:3] * w[2:3, :] + b_ref[...])
    o_ref[...] = _ln(jnp.tanh(h))


def _h0_call(x, W_in, b_in):
    return pl.pallas_call(
        _h0_body,
        grid=(N // _RN,),
        in_specs=[
            pl.BlockSpec((_RN, 3), lambda i: (i, 0)),
            pl.BlockSpec((3, H), lambda i: (0, 0)),
            pl.BlockSpec((H,), lambda i: (0,)),
        ],
        out_specs=pl.BlockSpec((_RN, H), lambda i: (i, 0)),
        out_shape=jax.ShapeDtypeStruct((N, H), F32),
    )(x, W_in, b_in)


def _proj_body(h_ref, wcat_ref, eb1_ref, hs_ref, hd_ref, hm_ref):
    p = _dot(h_ref[...], wcat_ref[...])
    hs_ref[...] = p[:, :H] + eb1_ref[...]
    hd_ref[...] = p[:, H:2 * H]
    hm_ref[0] = p[:, 2 * H:2 * H + 32]
    hm_ref[1] = p[:, 2 * H + 32:3 * H]


def _proj_call(h, Wcat, eb1):
    return pl.pallas_call(
        _proj_body,
        grid=(N // _RN,),
        in_specs=[
            pl.BlockSpec((_RN, H), lambda i: (i, 0)),
            pl.BlockSpec((H, 3 * H), lambda i: (0, 0)),
            pl.BlockSpec((H,), lambda i: (0,)),
        ],
        out_specs=[
            pl.BlockSpec((_RN, H), lambda i: (i, 0)),
            pl.BlockSpec((_RN, H), lambda i: (i, 0)),
            pl.BlockSpec((2, _RN, 32), lambda i: (0, i, 0)),
        ],
        out_shape=[
            jax.ShapeDtypeStruct((N, H), F32),
            jax.ShapeDtypeStruct((N, H), F32),
            jax.ShapeDtypeStruct((2, N, 32), F32),
        ],
    )(h, Wcat, eb1)


_SE = 7168


_SB = _SE // 2


def _edge_body(ep_ref, ew2_ref, eb2_ref, o_ref):
    i = pl.program_id(0)
    v = ep_ref[...]
    tl = jnp.tanh(_ln(v[:, :H]))
    tr = jnp.tanh(_ln(v[:, H:]))
    sl = _dot(tl, ew2_ref[...]) + eb2_ref[...]
    sr = _dot(tr, ew2_ref[...]) + eb2_ref[...]
    eg = jax.lax.broadcasted_iota(I32, (_SB, 1), 0) * 2 + i * _SE
    wl = jnp.where(eg < E, jax.nn.sigmoid(sl), 0.0)
    wr = jnp.where(eg + 1 < E, jax.nn.sigmoid(sr), 0.0)
    gr = _SE // _CE
    o_ref[...] = jnp.concatenate(
        [jnp.reshape(wl, (gr, H)), jnp.reshape(wr, (gr, H))], axis=1)


def _edge_call(epre2, eW2, eb2):
    return pl.pallas_call(
        _edge_body,
        grid=(EP // _SE,),
        in_specs=[
            pl.BlockSpec((_SB, 2 * H), lambda i: (i, 0)),
            pl.BlockSpec((H, 1), lambda i: (0, 0)),
            pl.BlockSpec((1,), lambda i: (0,)),
        ],
        out_specs=pl.BlockSpec((_SE // _CE, _CE), lambda i: (i, 0)),
        out_shape=jax.ShapeDtypeStruct((_G, _CE), F32),
    )(epre2, eW2, eb2)


def _node_body(h_ref, agg_ref, nwt_ref, nb1_ref, o_ref):
    h = h_ref[...]
    a2 = agg_ref[...]
    a = jnp.concatenate([a2[0], a2[1]], axis=-1)
    npre = _dot(h, nwt_ref[...]) + a + nb1_ref[...]
    o_ref[...] = jnp.tanh(_ln(npre)) + h


def _node_call(h, agg2, nW1t, nb1):
    return pl.pallas_call(
        _node_body,
        grid=(N // _RN,),
        in_specs=[
            pl.BlockSpec((_RN, H), lambda i: (i, 0)),
            pl.BlockSpec((2, _RN, 32), lambda i: (0, i, 0)),
            pl.BlockSpec((H, H), lambda i: (0, 0)),
            pl.BlockSpec((H,), lambda i: (0,)),
        ],
        out_specs=pl.BlockSpec((_RN, H), lambda i: (i, 0)),
        out_shape=jax.ShapeDtypeStruct((N, H), F32),
    )(h, agg2, nW1t, nb1)


def _head_body(p_ref, w0, b0, w1, b1, w2, b2, w3, b3, o_ref):
    ip = p_ref[0] + p_ref[1]
    y = jnp.tanh(_ln(_dot(ip, w0[...]) + b0[...]))
    y = jnp.tanh(_ln(_dot(y, w1[...]) + b1[...]))
    y = jnp.tanh(_ln(_dot(y, w2[...]) + b2[...]))
    o_ref[...] = _dot(y, w3[...]) + b3[...]


def _head_call(pool2, mW0, mb0, mW1, mb1, mW2, mb2, mW3, mb3):
    return pl.pallas_call(
        _head_body,
        out_shape=jax.ShapeDtypeStruct((B, 3), F32),
    )(pool2, mW0, mb0, mW1, mb1, mW2, mb2, mW3, mb3)


_MESH = dict(core_axis_name="c", subcore_axis_name="s",
             num_cores=NC, num_subcores=NS)
_CE = 128
_SK = 4
EP = 802816
_G = EP // _CE
_SSP = _G // _SK


def _zero16():
    return jnp.zeros((LANES,), F32)


def _sc_pair_body(hs_hbm, hd_hbm, src_hbm, dst_hbm, out_hbm,
                  idx_s, idx_d, rows_s, rows_d, rows_o, sem):
    c = lax.axis_index("c")
    s = lax.axis_index("s")
    wid = s * NC + c
    nw = NC * NS

    def step(t, _):
        ss = t * nw + wid
        g0 = ss * _SK
        pltpu.sync_copy(src_hbm.at[pl.ds(g0, _SK)], idx_s)
        pltpu.sync_copy(dst_hbm.at[pl.ds(g0, _SK)], idx_d)
        cps = [pltpu.async_copy(hs_hbm.at[idx_s.at[j]],
                                rows_s.at[pl.ds(j * _CE, _CE)], sem)
               for j in range(_SK)]
        cpd = [pltpu.async_copy(hd_hbm.at[idx_d.at[j]],
                                rows_d.at[pl.ds(j * _CE, _CE)], sem)
               for j in range(_SK)]
        for cp in cps + cpd:
            cp.wait()

        def rowf(r, _):
            for u in range(4):
                rr = r * 4 + u
                orow = r * 2 + u // 2
                oc = (u % 2) * H
                for q in range(H // LANES):
                    sl = pl.ds(q * LANES, LANES)
                    rows_o[orow, pl.ds(oc + q * LANES, LANES)] = (
                        rows_s[rr, sl] + rows_d[rr, sl])
            return 0

        lax.fori_loop(0, _SK * _CE // 4, rowf, 0)
        pltpu.sync_copy(rows_o, out_hbm.at[pl.ds(g0 * _CE // 2, _SK * _CE // 2)])
        return 0

    lax.fori_loop(0, _SSP // nw, step, 0)


def _pair_call(hs, hd, src2, dst2):
    mesh = plsc.VectorSubcoreMesh(**_MESH)
    f = pl.kernel(
        _sc_pair_body,
        out_type=jax.ShapeDtypeStruct((EP // 2, 2 * H), F32),
        mesh=mesh,
        compiler_params=pltpu.CompilerParams(use_tc_tiling_on_sc=False),
        scratch_types=[
            pltpu.VMEM((_SK, _CE), I32),
            pltpu.VMEM((_SK, _CE), I32),
            pltpu.VMEM((_SK * _CE, H), F32),
            pltpu.VMEM((_SK * _CE, H), F32),
            pltpu.VMEM((_SK * _CE // 2, 2 * H), F32),
            pltpu.SemaphoreType.DMA,
        ],
    )
    return f(hs, hd, src2, dst2)


_NSL = N // NS
_ZR = 125


def _sc_agg_body(hmlo_hbm, hmhi_hbm, src_hbm, dst_hbm, w_hbm, out_hbm,
                 idx, idxd, wv, rows, zbuf, spm, sem):
    c = lax.axis_index("c")
    s = lax.axis_index("s")

    def zf(r, _):
        for q in range(32 // LANES):
            zbuf[r, pl.ds(q * LANES, LANES)] = _zero16()
        return 0

    lax.fori_loop(0, _ZR, zf, 0)

    def zcopy(k, _):
        pltpu.sync_copy(zbuf, spm.at[pl.ds(s * _NSL + k * _ZR, _ZR)])
        return 0

    lax.fori_loop(0, _NSL // _ZR, zcopy, 0)
    plsc.subcore_barrier()

    def step(t, _):
        ss = t * NS + s
        g0 = ss * _SK
        pltpu.sync_copy(src_hbm.at[pl.ds(g0, _SK)], idx)
        pltpu.sync_copy(dst_hbm.at[pl.ds(g0, _SK)], idxd)
        pltpu.sync_copy(w_hbm.at[pl.ds(g0, _SK)], wv)

        @pl.when(c == 0)
        def _():
            cps = [pltpu.async_copy(hmlo_hbm.at[idx.at[j]],
                                    rows.at[pl.ds(j * _CE, _CE)], sem)
                   for j in range(_SK)]
            for cp in cps:
                cp.wait()

        @pl.when(c == 1)
        def _():
            cps = [pltpu.async_copy(hmhi_hbm.at[idx.at[j]],
                                    rows.at[pl.ds(j * _CE, _CE)], sem)
                   for j in range(_SK)]
            for cp in cps:
                cp.wait()

        def mf(m, _):
            for u in range(_SK):
                we = wv[u, pl.ds(m * LANES, LANES)]
                wo = wv[u, pl.ds(H + m * LANES, LANES)]
                for t in range(LANES):
                    re = u * _CE + m * 2 * LANES + 2 * t
                    for q in range(32 // LANES):
                        sl = pl.ds(q * LANES, LANES)
                        rows[re, sl] = rows[re, sl] * we[t]
                        rows[re + 1, sl] = rows[re + 1, sl] * wo[t]
            return 0

        lax.fori_loop(0, _CE // (2 * LANES), mf, 0)
        for j in range(_SK):
            pltpu.sync_copy(rows.at[pl.ds(j * _CE, _CE)],
                            spm.at[idxd.at[j]], add=True)
        return 0

    lax.fori_loop(0, _SSP // NS, step, 0)
    plsc.subcore_barrier()
    pltpu.sync_copy(spm.at[pl.ds(s * _NSL, _NSL)],
                    out_hbm.at[c, pl.ds(s * _NSL, _NSL)])


def _agg_call(hm_lo, hm_hi, src2, dst2, w2):
    mesh = plsc.VectorSubcoreMesh(**_MESH)
    f = pl.kernel(
        _sc_agg_body,
        out_type=jax.ShapeDtypeStruct((2, N, 32), F32),
        mesh=mesh,
        compiler_params=pltpu.CompilerParams(use_tc_tiling_on_sc=False),
        scratch_types=[
            pltpu.VMEM((_SK, _CE), I32),
            pltpu.VMEM((_SK, _CE), I32),
            pltpu.VMEM((_SK, _CE), F32),
            pltpu.VMEM((_SK * _CE, 32), F32),
            pltpu.VMEM((_ZR, 32), F32),
            pltpu.VMEM_SHARED((N, 32), F32),
            pltpu.SemaphoreType.DMA,
        ],
    )
    return f(hm_lo, hm_hi, src2, dst2, w2)


_CP = 80
_PCHUNKS = N // _CP
_BSL = B // NS


def _sc_pool_body(h_hbm, bat_hbm, out_hbm, idx, rows, zbuf, spm, sem):
    c = lax.axis_index("c")
    s = lax.axis_index("s")
    wid = s * NC + c
    nw = NC * NS

    def zf(r, _):
        for q in range(H // LANES):
            zbuf[r, pl.ds(q * LANES, LANES)] = _zero16()
        return 0

    lax.fori_loop(0, _BSL, zf, 0)
    pltpu.sync_copy(zbuf, spm.at[pl.ds(s * _BSL, _BSL)])
    plsc.subcore_barrier()

    def chunk(k, _):
        ch = k * nw + wid

        @pl.when(ch < _PCHUNKS)
        def _():
            base = ch * _CP
            pltpu.sync_copy(bat_hbm.at[pl.ds(base, _CP)], idx)
            pltpu.sync_copy(h_hbm.at[pl.ds(base, _CP)], rows)
            pltpu.sync_copy(rows, spm.at[idx], add=True)

        return 0

    lax.fori_loop(0, (_PCHUNKS + NC * NS - 1) // (NC * NS), chunk, 0)
    plsc.subcore_barrier()
    pltpu.sync_copy(spm.at[pl.ds(s * _BSL, _BSL)],
                    out_hbm.at[c, pl.ds(s * _BSL, _BSL)])


def _pool_call(h, bat):
    mesh = plsc.VectorSubcoreMesh(**_MESH)
    f = pl.kernel(
        _sc_pool_body,
        out_type=jax.ShapeDtypeStruct((2, B, H), F32),
        mesh=mesh,
        compiler_params=pltpu.CompilerParams(use_tc_tiling_on_sc=False),
        scratch_types=[
            pltpu.VMEM((_CP,), I32),
            pltpu.VMEM((_CP, H), F32),
            pltpu.VMEM((_BSL, H), F32),
            pltpu.VMEM_SHARED((B, H), F32),
            pltpu.SemaphoreType.DMA,
        ],
    )
    return f(h, bat)


def kernel(x, edge_index, batch, batch_size,
           W_in, b_in, eW1, eb1, eW2, eb2, nW1, nb1,
           mW0, mb0, mW1, mb1, mW2, mb2, mW3, mb3):
    src = edge_index[0].astype(I32)
    dst = edge_index[1].astype(I32)
    pad = jnp.zeros((EP - E,), I32)
    src2 = jnp.concatenate([src, pad]).reshape(_G, _CE)
    dst2 = jnp.concatenate([dst, pad]).reshape(_G, _CE)
    bat = batch.astype(I32)
    Wcat = jnp.concatenate([eW1[:H], eW1[H:], nW1[H:]], axis=1)
    nW1t = nW1[:H]

    h = _h0_call(x, W_in, b_in)
    for _ in range(N_ITERS):
        hs, hd, hm2 = _proj_call(h, Wcat, eb1)
        epre2 = _pair_call(hs, hd, src2, dst2)
        w2 = _edge_call(epre2, eW2, eb2)
        agg2 = _agg_call(hm2[0], hm2[1], src2, dst2, w2)
        h = _node_call(h, agg2, nW1t, nb1)

    pool2 = _pool_call(h, bat)
    return _head_call(pool2, mW0, mb0, mW1, mb1, mW2, mb2, mW3, mb3)
